# baseline (speedup 1.0000x reference)
import functools

import jax
import jax.numpy as jnp
from jax import lax
from jax.experimental import pallas as pl
from jax.experimental.pallas import tpu as pltpu
from jax.experimental.pallas import tpu_sc as plsc

VOCAB = 8192
D = 8192
B = 16384
FRAC = 4
PD = D // FRAC
NC = 2
NS = 16
NW = NC * NS
BPW = B // NW
CHUNK = 8
NCH = BPW // CHUNK
NU = NCH * FRAC
NBUF = 4
NGRP = NU // NBUF


def _gather_body(idx_hbm, table_hbm, out_hbm, idx_v, rows_v, gsems, ssems):
    wid = lax.axis_index("s") * NC + lax.axis_index("c")
    base = wid * BPW
    pltpu.sync_copy(idx_hbm.at[wid], idx_v)

    def gather(u, b):
        g, h = u // FRAC, u % FRAC
        return pltpu.make_async_copy(
            table_hbm.at[idx_v.at[g], pl.ds(h * PD, PD)],
            rows_v.at[b], gsems.at[b])

    def scatter(u, b):
        g, h = u // FRAC, u % FRAC
        return pltpu.make_async_copy(
            rows_v.at[b],
            out_hbm.at[pl.ds(base + g * CHUNK, CHUNK), pl.ds(h * PD, PD)],
            ssems.at[b])

    for b in range(NBUF):
        gather(b, b).start()

    def body(t, carry):
        for b in range(NBUF):
            u = t * NBUF + b
            gather(u, b).wait()
            scatter(u, b).start()
            nxt = u + NBUF

            @pl.when(nxt < NU)
            def _():
                scatter(u, b).wait()
                gather(nxt, b).start()

        return carry

    lax.fori_loop(0, NGRP, body, 0)
    for b in range(NBUF):
        scatter(NU - NBUF + b, b).wait()


@jax.jit
def _gather(idx_r, table):
    mesh = plsc.VectorSubcoreMesh(core_axis_name="c", subcore_axis_name="s")
    k = functools.partial(
        pl.kernel,
        mesh=mesh,
        out_type=jax.ShapeDtypeStruct((B, D), jnp.float32),
        scratch_types=[
            pltpu.VMEM((NCH, CHUNK), jnp.int32),
            pltpu.VMEM((NBUF, CHUNK, PD), jnp.float32),
            pltpu.SemaphoreType.DMA((NBUF,)),
            pltpu.SemaphoreType.DMA((NBUF,)),
        ],
    )(_gather_body)
    return k(idx_r, table)


def kernel(idx, table):
    idx_r = jnp.reshape(idx.astype(jnp.int32), (NW, NCH, CHUNK))
    return _gather(idx_r, table)


# speedup vs baseline: 1.0011x; 1.0011x over previous
import functools

import jax
import jax.numpy as jnp
from jax import lax
from jax.experimental import pallas as pl
from jax.experimental.pallas import tpu as pltpu
from jax.experimental.pallas import tpu_sc as plsc

VOCAB = 8192
D = 8192
B = 16384
FRAC = 4
PD = D // FRAC
NC = 2
NS = 16
NW = NC * NS
BPW = B // NW
CHUNK = 8
NCH = BPW // CHUNK
NU = NCH * FRAC
NBUF = 4
NGRP = NU // NBUF


def _gather_body(idx_hbm, table_hbm, out_hbm, idx_v, rows_v, gsems, ssems):
    wid = lax.axis_index("s") * NC + lax.axis_index("c")
    base = wid * BPW
    pltpu.sync_copy(idx_hbm.at[wid], idx_v)

    def gather(u, b):
        g, h = u // FRAC, u % FRAC
        return pltpu.make_async_copy(
            table_hbm.at[idx_v.at[g], pl.ds(h * PD, PD)],
            rows_v.at[b], gsems.at[b])

    def scatter(u, b):
        g, h = u // FRAC, u % FRAC
        return pltpu.make_async_copy(
            rows_v.at[b],
            out_hbm.at[pl.ds(base + g * CHUNK, CHUNK), pl.ds(h * PD, PD)],
            ssems.at[b])

    for b in range(NBUF):
        gather(b, b).start()

    def step(u, b):
        bp = (b - 1) % NBUF
        gather(u, b).wait()
        scatter(u, b).start()
        scatter(u - 1, bp).wait()
        nxt = u - 1 + NBUF

        @pl.when(nxt < NU)
        def _():
            gather(nxt, bp).start()

    gather(0, 0).wait()
    scatter(0, 0).start()

    def body(t, carry):
        for b in range(NBUF):
            step(t * NBUF + b + 1, (b + 1) % NBUF)
        return carry

    lax.fori_loop(0, (NU - 1) // NBUF, body, 0)
    for i in range(NU - 1 - ((NU - 1) // NBUF) * NBUF):
        step((NU - 1) // NBUF * NBUF + 1 + i, (i + 1) % NBUF)
    scatter(NU - 1, (NU - 1) % NBUF).wait()


@jax.jit
def _gather(idx_r, table):
    mesh = plsc.VectorSubcoreMesh(core_axis_name="c", subcore_axis_name="s")
    k = functools.partial(
        pl.kernel,
        mesh=mesh,
        out_type=jax.ShapeDtypeStruct((B, D), jnp.float32),
        scratch_types=[
            pltpu.VMEM((NCH, CHUNK), jnp.int32),
            pltpu.VMEM((NBUF, CHUNK, PD), jnp.float32),
            pltpu.SemaphoreType.DMA((NBUF,)),
            pltpu.SemaphoreType.DMA((NBUF,)),
        ],
    )(_gather_body)
    return k(idx_r, table)


def kernel(idx, table):
    idx_r = jnp.reshape(idx.astype(jnp.int32), (NW, NCH, CHUNK))
    return _gather(idx_r, table)
